# baseline (speedup 1.0000x reference)
import functools

import jax
import jax.numpy as jnp
from jax import lax
from jax.experimental import pallas as pl
from jax.experimental.pallas import tpu as pltpu
from jax.experimental.pallas import tpu_sc as plsc

_NC = 2
_NS = 16
_NW = _NC * _NS
_LANES = 16
_CHUNK = 100
_NBUF = 4


def _position_encoding(seq_len, hidden, start, dtype):
    power = jnp.arange(0, hidden, 2, dtype=dtype) / hidden
    divisor = 10000.0 ** power
    seqpos = jnp.arange(start, seq_len + start, dtype=dtype)
    index = seqpos[:, None] / divisor[None, :]
    pe = jnp.stack((jnp.sin(index), jnp.cos(index)), axis=-1)
    return pe.reshape(seq_len, hidden)


def _body(idx_hbm, table_hbm, pe_hbm, out_hbm, idx_v, rows_v, pe_v, gsem, osem):
    nb = idx_hbm.shape[0] // _NW
    seq = pe_hbm.shape[0]
    nchunk = seq // _CHUNK
    wid = lax.axis_index("s") * _NC + lax.axis_index("c")
    base = wid * nb

    pltpu.sync_copy(pe_hbm, pe_v)
    pltpu.sync_copy(idx_hbm.at[pl.ds(base, nb)], idx_v)

    def stage(i):
        buf = lax.rem(i, _NBUF)

        @pl.when(i >= _NBUF)
        def _drain_scatter():
            pltpu.make_async_copy(
                rows_v.at[buf], out_hbm.at[base + i - _NBUF], osem.at[buf]
            ).wait()

        for j in range(nchunk):
            pltpu.async_copy(
                table_hbm.at[idx_v.at[i, j]],
                rows_v.at[buf, pl.ds(j * _CHUNK, _CHUNK)],
                gsem.at[buf],
            )

    def process(j):
        buf = lax.rem(j, _NBUF)
        for j2 in range(nchunk):
            pltpu.make_async_copy(
                table_hbm.at[idx_v.at[j, j2]],
                rows_v.at[buf, pl.ds(j2 * _CHUNK, _CHUNK)],
                gsem.at[buf],
            ).wait()

        def row_body(r4, carry):
            for dr in range(4):
                r = r4 * 4 + dr
                for c in range(4):
                    sl = pl.ds(c * _LANES, _LANES)
                    rows_v[buf, r, sl] = rows_v[buf, r, sl] + pe_v[r, sl]
            return carry

        lax.fori_loop(0, seq // 4, row_body, 0)
        pltpu.async_copy(rows_v.at[buf], out_hbm.at[base + j], osem.at[buf])

    def batch_body(i, carry):
        stage(i)

        @pl.when(i >= 1)
        def _proc():
            process(i - 1)

        return carry

    lax.fori_loop(0, nb, batch_body, 0)
    process(nb - 1)

    def drain_body(j, carry):
        buf = lax.rem(j, _NBUF)
        pltpu.make_async_copy(
            rows_v.at[buf], out_hbm.at[base + j], osem.at[buf]
        ).wait()
        return carry

    lax.fori_loop(nb - _NBUF, nb, drain_body, 0)


def kernel(inputs, embedding_table):
    batch, seq = inputs.shape
    _, embed = embedding_table.shape
    pe = _position_encoding(seq, embed, 1, embedding_table.dtype)
    idx = inputs.reshape(batch, seq // _CHUNK, _CHUNK)

    mesh = plsc.VectorSubcoreMesh(
        core_axis_name="c", subcore_axis_name="s", num_cores=_NC, num_subcores=_NS
    )
    run = pl.kernel(
        _body,
        out_type=jax.ShapeDtypeStruct((batch, seq, embed), embedding_table.dtype),
        mesh=mesh,
        scratch_types=[
            pltpu.VMEM((batch // _NW, seq // _CHUNK, _CHUNK), jnp.int32),
            pltpu.VMEM((_NBUF, seq, embed), jnp.float32),
            pltpu.VMEM((seq, embed), jnp.float32),
            pltpu.SemaphoreType.DMA((_NBUF,)),
            pltpu.SemaphoreType.DMA((_NBUF,)),
        ],
        compiler_params=pltpu.CompilerParams(use_tc_tiling_on_sc=False),
    )
    return run(idx, embedding_table, pe)


# speedup vs baseline: 1.0221x; 1.0221x over previous
import functools

import jax
import jax.numpy as jnp
from jax import lax
from jax.experimental import pallas as pl
from jax.experimental.pallas import tpu as pltpu
from jax.experimental.pallas import tpu_sc as plsc

_NC = 2
_NS = 16
_NW = _NC * _NS
_CHUNK = 100
_NBUF = 4


def _position_encoding(seq_len, hidden, start, dtype):
    power = jnp.arange(0, hidden, 2, dtype=dtype) / hidden
    divisor = 10000.0 ** power
    seqpos = jnp.arange(start, seq_len + start, dtype=dtype)
    index = seqpos[:, None] / divisor[None, :]
    pe = jnp.stack((jnp.sin(index), jnp.cos(index)), axis=-1)
    return pe.reshape(seq_len, hidden)


def _body(idx_hbm, table_hbm, pe_hbm, out_hbm, idx_v, rows_v, isem, gsem, osem):
    nb = idx_hbm.shape[0] // _NW
    seq = pe_hbm.shape[0]
    nchunk = seq // _CHUNK
    wid = lax.axis_index("s") * _NC + lax.axis_index("c")
    base = wid * nb

    pltpu.sync_copy(idx_hbm.at[pl.ds(base, nb)], idx_v)

    def stage_a(i):
        buf = lax.rem(i, _NBUF)

        @pl.when(i >= _NBUF)
        def _drain_scatter():
            pltpu.make_async_copy(
                rows_v.at[buf], out_hbm.at[base + i - _NBUF], osem.at[buf]
            ).wait()

        pltpu.async_copy(pe_hbm, rows_v.at[buf], isem.at[buf])

    def stage_b(i):
        buf = lax.rem(i, _NBUF)
        pltpu.make_async_copy(pe_hbm, rows_v.at[buf], isem.at[buf]).wait()
        for j in range(nchunk):
            pltpu.async_copy(
                table_hbm.at[idx_v.at[i, j]],
                rows_v.at[buf, pl.ds(j * _CHUNK, _CHUNK)],
                gsem.at[buf],
                add=True,
            )

    def stage_c(i):
        buf = lax.rem(i, _NBUF)
        for j in range(nchunk):
            pltpu.make_async_copy(
                table_hbm.at[idx_v.at[i, j]],
                rows_v.at[buf, pl.ds(j * _CHUNK, _CHUNK)],
                gsem.at[buf],
            ).wait()
        pltpu.async_copy(rows_v.at[buf], out_hbm.at[base + i], osem.at[buf])

    def loop_body(i, carry):
        @pl.when(i < nb)
        def _a():
            stage_a(i)

        @pl.when(jnp.logical_and(i >= 1, i <= nb))
        def _b():
            stage_b(i - 1)

        @pl.when(i >= 2)
        def _c():
            stage_c(i - 2)

        return carry

    lax.fori_loop(0, nb + 2, loop_body, 0)

    def drain_body(i, carry):
        buf = lax.rem(i, _NBUF)
        pltpu.make_async_copy(
            rows_v.at[buf], out_hbm.at[base + i], osem.at[buf]
        ).wait()
        return carry

    lax.fori_loop(nb - _NBUF, nb, drain_body, 0)


def kernel(inputs, embedding_table):
    batch, seq = inputs.shape
    _, embed = embedding_table.shape
    pe = _position_encoding(seq, embed, 1, embedding_table.dtype)
    idx = inputs.reshape(batch, seq // _CHUNK, _CHUNK)

    mesh = plsc.VectorSubcoreMesh(
        core_axis_name="c", subcore_axis_name="s", num_cores=_NC, num_subcores=_NS
    )
    run = pl.kernel(
        _body,
        out_type=jax.ShapeDtypeStruct((batch, seq, embed), embedding_table.dtype),
        mesh=mesh,
        scratch_types=[
            pltpu.VMEM((batch // _NW, seq // _CHUNK, _CHUNK), jnp.int32),
            pltpu.VMEM((_NBUF, seq, embed), jnp.float32),
            pltpu.SemaphoreType.DMA((_NBUF,)),
            pltpu.SemaphoreType.DMA((_NBUF,)),
            pltpu.SemaphoreType.DMA((_NBUF,)),
        ],
        compiler_params=pltpu.CompilerParams(use_tc_tiling_on_sc=False),
    )
    return run(idx, embedding_table, pe)
